# baseline (speedup 1.0000x reference)
import jax
import jax.numpy as jnp
from jax.experimental import pallas as pl
from jax.experimental.pallas import tpu as pltpu

_BN = 1024
_C = 128


def _tmk_block(x_ref, ptsb_ref, c_ref, out_ref):
    D = x_ref.shape[1]
    M = c_ref.shape[0]
    c = c_ref[...]
    for j in range(_BN // _C):
        xct = x_ref[j * _C : (j + 1) * _C, :].T
        acc = None
        for d in range(D):
            xr = xct[d : d + 1, :]
            pb = ptsb_ref[d * M : (d + 1) * M, :]
            t = jnp.abs(pb - xr)
            acc = t if acc is None else acc + t
        kt = jnp.exp(-acc)
        out_ref[j * _C : (j + 1) * _C, :] = jax.lax.dot_general(
            kt, c, (((0,), (0,)), ((), ())), preferred_element_type=jnp.float32
        )


def kernel(input, pts_set, chol_inv):
    N, D = input.shape
    M = pts_set.shape[0]
    pts_b = jnp.broadcast_to(pts_set.T[:, :, None], (D, M, _C)).reshape(D * M, _C)
    return pl.pallas_call(
        _tmk_block,
        grid=(N // _BN,),
        in_specs=[
            pl.BlockSpec((_BN, D), lambda i: (i, 0)),
            pl.BlockSpec((D * M, _C), lambda i: (0, 0)),
            pl.BlockSpec((M, M), lambda i: (0, 0)),
        ],
        out_specs=pl.BlockSpec((_BN, M), lambda i: (i, 0)),
        out_shape=jax.ShapeDtypeStruct((N, M), jnp.float32),
        compiler_params=pltpu.CompilerParams(
            dimension_semantics=("parallel",),
        ),
    )(input, pts_b, chol_inv)


# speedup vs baseline: 1.9549x; 1.9549x over previous
import jax
import jax.numpy as jnp
from jax.experimental import pallas as pl

_BN = 2048
_C = 128


def _tmk_block(xt_ref, ptsb_ref, c_ref, out_ref):
    D = xt_ref.shape[0]
    M = c_ref.shape[0]
    c = c_ref[...]
    for j in range(_BN // _C):
        acc = None
        for d in range(D):
            xr = xt_ref[d : d + 1, j * _C : (j + 1) * _C]
            pb = ptsb_ref[d * M : (d + 1) * M, :]
            t = jnp.abs(pb - xr)
            acc = t if acc is None else acc + t
        kt = jnp.exp(-acc)
        out_ref[j * _C : (j + 1) * _C, :] = jax.lax.dot_general(
            kt, c, (((0,), (0,)), ((), ())), preferred_element_type=jnp.float32
        )


def kernel(input, pts_set, chol_inv):
    N, D = input.shape
    M = pts_set.shape[0]
    xt = input.T
    pts_b = jnp.broadcast_to(pts_set.T[:, :, None], (D, M, _C)).reshape(D * M, _C)
    return pl.pallas_call(
        _tmk_block,
        grid=(N // _BN,),
        in_specs=[
            pl.BlockSpec((D, _BN), lambda i: (0, i)),
            pl.BlockSpec((D * M, _C), lambda i: (0, 0)),
            pl.BlockSpec((M, M), lambda i: (0, 0)),
        ],
        out_specs=pl.BlockSpec((_BN, M), lambda i: (i, 0)),
        out_shape=jax.ShapeDtypeStruct((N, M), jnp.float32),
    )(xt, pts_b, chol_inv)


# speedup vs baseline: 2.0900x; 1.0691x over previous
import jax
import jax.numpy as jnp
from jax.experimental import pallas as pl

_BN = 4096
_C = 128


def _tmk_block(xt_ref, ptsb_ref, c_ref, out_ref):
    D = xt_ref.shape[0]
    M = c_ref.shape[0]
    c = c_ref[...]
    for j in range(_BN // _C):
        acc = None
        for d in range(D):
            xr = xt_ref[d : d + 1, j * _C : (j + 1) * _C]
            pb = ptsb_ref[d * M : (d + 1) * M, :]
            t = jnp.abs(pb - xr)
            acc = t if acc is None else acc + t
        kt = jnp.exp(-acc)
        out_ref[j * _C : (j + 1) * _C, :] = jax.lax.dot_general(
            kt, c, (((0,), (0,)), ((), ())), preferred_element_type=jnp.float32
        )


def kernel(input, pts_set, chol_inv):
    N, D = input.shape
    M = pts_set.shape[0]
    xt = input.T
    pts_b = jnp.broadcast_to(pts_set.T[:, :, None], (D, M, _C)).reshape(D * M, _C)
    return pl.pallas_call(
        _tmk_block,
        grid=(N // _BN,),
        in_specs=[
            pl.BlockSpec((D, _BN), lambda i: (0, i)),
            pl.BlockSpec((D * M, _C), lambda i: (0, 0)),
            pl.BlockSpec((M, M), lambda i: (0, 0)),
        ],
        out_specs=pl.BlockSpec((_BN, M), lambda i: (i, 0)),
        out_shape=jax.ShapeDtypeStruct((N, M), jnp.float32),
    )(xt, pts_b, chol_inv)


# speedup vs baseline: 2.1457x; 1.0266x over previous
import jax
import jax.numpy as jnp
from jax.experimental import pallas as pl

_BN = 8192
_C = 128


def _tmk_block(xt_ref, ptsb_ref, c_ref, out_ref):
    D = xt_ref.shape[0]
    M = c_ref.shape[0]
    c = c_ref[...]
    for j in range(_BN // _C):
        acc = None
        for d in range(D):
            xr = xt_ref[d : d + 1, j * _C : (j + 1) * _C]
            pb = ptsb_ref[d * M : (d + 1) * M, :]
            t = jnp.abs(pb - xr)
            acc = t if acc is None else acc + t
        kt = jnp.exp(-acc)
        out_ref[j * _C : (j + 1) * _C, :] = jax.lax.dot_general(
            kt, c, (((0,), (0,)), ((), ())), preferred_element_type=jnp.float32
        )


def kernel(input, pts_set, chol_inv):
    N, D = input.shape
    M = pts_set.shape[0]
    xt = input.T
    pts_b = jnp.broadcast_to(pts_set.T[:, :, None], (D, M, _C)).reshape(D * M, _C)
    return pl.pallas_call(
        _tmk_block,
        grid=(N // _BN,),
        in_specs=[
            pl.BlockSpec((D, _BN), lambda i: (0, i)),
            pl.BlockSpec((D * M, _C), lambda i: (0, 0)),
            pl.BlockSpec((M, M), lambda i: (0, 0)),
        ],
        out_specs=pl.BlockSpec((_BN, M), lambda i: (i, 0)),
        out_shape=jax.ShapeDtypeStruct((N, M), jnp.float32),
    )(xt, pts_b, chol_inv)


# speedup vs baseline: 2.1461x; 1.0002x over previous
import jax
import jax.numpy as jnp
from jax.experimental import pallas as pl

_BN = 16384
_C = 128


def _tmk_block(xt_ref, ptsb_ref, c_ref, out_ref):
    D = xt_ref.shape[0]
    M = c_ref.shape[0]
    c = c_ref[...]
    for j in range(_BN // _C):
        acc = None
        for d in range(D):
            xr = xt_ref[d : d + 1, j * _C : (j + 1) * _C]
            pb = ptsb_ref[d * M : (d + 1) * M, :]
            t = jnp.abs(pb - xr)
            acc = t if acc is None else acc + t
        kt = jnp.exp(-acc)
        out_ref[j * _C : (j + 1) * _C, :] = jax.lax.dot_general(
            kt, c, (((0,), (0,)), ((), ())), preferred_element_type=jnp.float32
        )


def kernel(input, pts_set, chol_inv):
    N, D = input.shape
    M = pts_set.shape[0]
    xt = input.T
    pts_b = jnp.broadcast_to(pts_set.T[:, :, None], (D, M, _C)).reshape(D * M, _C)
    return pl.pallas_call(
        _tmk_block,
        grid=(N // _BN,),
        in_specs=[
            pl.BlockSpec((D, _BN), lambda i: (0, i)),
            pl.BlockSpec((D * M, _C), lambda i: (0, 0)),
            pl.BlockSpec((M, M), lambda i: (0, 0)),
        ],
        out_specs=pl.BlockSpec((_BN, M), lambda i: (i, 0)),
        out_shape=jax.ShapeDtypeStruct((N, M), jnp.float32),
    )(xt, pts_b, chol_inv)
